# baseline (speedup 1.0000x reference)
import functools

import jax
import jax.numpy as jnp
from jax import lax
from jax.experimental import pallas as pl
from jax.experimental.pallas import tpu as pltpu
from jax.experimental.pallas import tpu_sc as plsc

_VOCAB = 1000000
_DIM = 64
_B = 16384
_CTX = 20
_LANES = 16

_NW = 32
_TASKS = 2 * _B
_TPW = _TASKS // _NW
_CB = 4
_ROWS = _CB * _CTX
_NCHUNK = _TPW // _CB
_UBLK = 32
_NUB = _TPW // _UBLK
_CPU = _UBLK // (2 * _CB)
_MROWS = _TPW // 8


def _sc_body(vdiv_hbm, udiv_hbm, wtab_hbm, p_hbm,
             vdiv_v, udiv_v, ubuf0, ubuf1,
             vbuf0, vbuf1, pbuf, vsem0, vsem1, usem0, usem1, psem):
    cid = lax.axis_index("c")
    sid = lax.axis_index("s")
    wid = sid * 2 + cid

    pltpu.sync_copy(vdiv_hbm.at[wid], vdiv_v)
    pltpu.sync_copy(udiv_hbm.at[wid], udiv_v)

    pltpu.async_copy(wtab_hbm.at[udiv_v.at[0]], ubuf0, usem0)
    pltpu.async_copy(wtab_hbm.at[vdiv_v.at[0]], vbuf0, vsem0)

    sign = jnp.where(wid < _NW // 2, 1.0, -1.0).astype(jnp.float32)
    scale = sign * (1.0 / _CTX)

    vbufs = (vbuf0, vbuf1)
    vsems = (vsem0, vsem1)
    ubufs = (ubuf0, ubuf1)
    usems = (usem0, usem1)

    @pl.loop(0, _NUB // 2)
    def _outer(uu):
        for u2 in range(2):
            ublk = uu * 2 + u2

            pltpu.make_async_copy(wtab_hbm.at[udiv_v.at[ublk]],
                                  ubufs[u2], usems[u2]).wait()

            @pl.when(ublk + 1 < _NUB)
            def _():
                nb = jnp.minimum(ublk + 1, _NUB - 1)
                pltpu.async_copy(wtab_hbm.at[udiv_v.at[nb]],
                                 ubufs[1 - u2], usems[1 - u2])

            ubuf = ubufs[u2]

            @pl.loop(0, _CPU)
            def _inner(cc):
                mg = ublk * _CPU + cc
                for b in range(2):
                    chunk = mg * 2 + b
                    nxt = chunk + 1

                    @pl.when(nxt < _NCHUNK)
                    def _():
                        nrow = jnp.minimum(nxt, _NCHUNK - 1)
                        pltpu.async_copy(wtab_hbm.at[vdiv_v.at[nrow]],
                                         vbufs[1 - b], vsems[1 - b])

                    pltpu.make_async_copy(wtab_hbm.at[vdiv_v.at[chunk]],
                                          vbufs[b], vsems[b]).wait()

                    buf = vbufs[b]
                    for t in range(_CB):
                        lrow = (cc * 2 + b) * _CB + t
                        p = None
                        for d in range(_DIM // _LANES):
                            sl = pl.ds(d * _LANES, _LANES)
                            acc = None
                            for c in range(_CTX):
                                x = buf[t * _CTX + c, sl]
                                acc = x if acc is None else acc + x
                            urow = ubuf[lrow,
                                        pl.ds(_DIM + d * _LANES, _LANES)]
                            term = acc * urow
                            p = term if p is None else p + term
                        pbuf[mg, pl.ds((_CB * b + t) * _LANES, _LANES)] = \
                            p * scale
                pltpu.async_copy(pbuf.at[mg], p_hbm.at[wid, mg], psem)

    @pl.loop(0, _MROWS)
    def _drain(i):
        pltpu.make_async_copy(pbuf.at[0], p_hbm.at[wid, 0], psem).wait()


@functools.cache
def _sc_pdots():
    return pl.kernel(
        _sc_body,
        out_type=jax.ShapeDtypeStruct((_NW, _MROWS, 8 * _LANES),
                                      jnp.float32),
        mesh=plsc.VectorSubcoreMesh(core_axis_name="c", subcore_axis_name="s",
                                    num_cores=2, num_subcores=16),
        compiler_params=pltpu.CompilerParams(use_tc_tiling_on_sc=True),
        scratch_types=[
            pltpu.VMEM((_NCHUNK, _ROWS), jnp.int32),
            pltpu.VMEM((_NUB, _UBLK), jnp.int32),
            pltpu.VMEM((_UBLK, 2 * _DIM), jnp.float32),
            pltpu.VMEM((_UBLK, 2 * _DIM), jnp.float32),
            pltpu.VMEM((_ROWS, 2 * _DIM), jnp.float32),
            pltpu.VMEM((_ROWS, 2 * _DIM), jnp.float32),
            pltpu.VMEM((_MROWS, 8 * _LANES), jnp.float32),
            pltpu.SemaphoreType.DMA,
            pltpu.SemaphoreType.DMA,
            pltpu.SemaphoreType.DMA,
            pltpu.SemaphoreType.DMA,
            pltpu.SemaphoreType.DMA,
        ],
    )


_WB = 4000


def _concat_body(v_ref, u_ref, o_ref):
    o_ref[:, 0:_DIM] = v_ref[...]
    o_ref[:, _DIM:2 * _DIM] = u_ref[...]


_concat_call = pl.pallas_call(
    _concat_body,
    grid=(_VOCAB // _WB,),
    in_specs=[
        pl.BlockSpec((_WB, _DIM), lambda i: (i, 0)),
        pl.BlockSpec((_WB, _DIM), lambda i: (i, 0)),
    ],
    out_specs=pl.BlockSpec((_WB, 2 * _DIM), lambda i: (i, 0)),
    out_shape=jax.ShapeDtypeStruct((_VOCAB, 2 * _DIM), jnp.float32),
)


def _loss_body(p_ref, out_ref):
    x = p_ref[...]
    j = lax.broadcasted_iota(jnp.int32, (8 * _LANES, 8), 0)
    t = lax.broadcasted_iota(jnp.int32, (8 * _LANES, 8), 1)
    sel = (j // _LANES == t).astype(jnp.float32)
    z = jnp.dot(x, sel, preferred_element_type=jnp.float32)
    out_ref[0, 0] = -jnp.sum(jax.nn.log_sigmoid(z))


_loss_call = pl.pallas_call(
    _loss_body,
    out_shape=jax.ShapeDtypeStruct((1, 1), jnp.float32),
    out_specs=pl.BlockSpec(memory_space=pltpu.SMEM),
)


def kernel(pos_v, pos_u, neg_v, neg_u, v_table, u_table):
    vidx = jnp.concatenate([pos_v.astype(jnp.int32).reshape(-1),
                            neg_v.astype(jnp.int32).reshape(-1)])
    uidx = jnp.concatenate([pos_u.astype(jnp.int32),
                            neg_u.astype(jnp.int32)])
    vdiv = vidx.reshape(_NW, _NCHUNK, _ROWS)
    udiv = uidx.reshape(_NW, _NUB, _UBLK)
    wtab = _concat_call(v_table, u_table)
    p = _sc_pdots()(vdiv, udiv, wtab)
    loss = _loss_call(p.reshape(_TASKS // 8, 8 * _LANES))
    return loss[0, 0]


# speedup vs baseline: 1.1363x; 1.1363x over previous
import functools

import jax
import jax.numpy as jnp
from jax import lax
from jax.experimental import pallas as pl
from jax.experimental.pallas import tpu as pltpu
from jax.experimental.pallas import tpu_sc as plsc

_VOCAB = 1000000
_DIM = 64
_B = 16384
_CTX = 20
_LANES = 16

_NW = 32
_TASKS = 2 * _B
_TPW = _TASKS // _NW
_CB = 4
_ROWS = _CB * _CTX
_NCHUNK = _TPW // _CB
_UBLK = 32
_NUB = _TPW // _UBLK
_CPU = _UBLK // (2 * _CB)
_MROWS = _TPW // 8


def _sc_body(vdiv_hbm, udiv_hbm, vtab_hbm, utab_hbm, p_hbm,
             vdiv_v, udiv_v, ubuf0, ubuf1,
             vbuf0, vbuf1, pbuf, vsem0, vsem1, usem0, usem1, psem):
    cid = lax.axis_index("c")
    sid = lax.axis_index("s")
    wid = sid * 2 + cid

    pltpu.sync_copy(vdiv_hbm.at[wid], vdiv_v)
    pltpu.sync_copy(udiv_hbm.at[wid], udiv_v)

    pltpu.async_copy(utab_hbm.at[udiv_v.at[0]], ubuf0, usem0)
    pltpu.async_copy(vtab_hbm.at[vdiv_v.at[0]], vbuf0, vsem0)

    sign = jnp.where(wid < _NW // 2, 1.0, -1.0).astype(jnp.float32)
    scale = sign * (1.0 / _CTX)

    vbufs = (vbuf0, vbuf1)
    vsems = (vsem0, vsem1)
    ubufs = (ubuf0, ubuf1)
    usems = (usem0, usem1)

    @pl.loop(0, _NUB // 2)
    def _outer(uu):
        for u2 in range(2):
            ublk = uu * 2 + u2

            pltpu.make_async_copy(utab_hbm.at[udiv_v.at[ublk]],
                                  ubufs[u2], usems[u2]).wait()

            @pl.when(ublk + 1 < _NUB)
            def _():
                nb = jnp.minimum(ublk + 1, _NUB - 1)
                pltpu.async_copy(utab_hbm.at[udiv_v.at[nb]],
                                 ubufs[1 - u2], usems[1 - u2])

            ubuf = ubufs[u2]

            @pl.loop(0, _CPU)
            def _inner(cc):
                mg = ublk * _CPU + cc
                for b in range(2):
                    chunk = mg * 2 + b
                    nxt = chunk + 1

                    @pl.when(nxt < _NCHUNK)
                    def _():
                        nrow = jnp.minimum(nxt, _NCHUNK - 1)
                        pltpu.async_copy(vtab_hbm.at[vdiv_v.at[nrow]],
                                         vbufs[1 - b], vsems[1 - b])

                    pltpu.make_async_copy(vtab_hbm.at[vdiv_v.at[chunk]],
                                          vbufs[b], vsems[b]).wait()

                    buf = vbufs[b]
                    for t in range(_CB):
                        lrow = (cc * 2 + b) * _CB + t
                        p = None
                        for d in range(_DIM // _LANES):
                            sl = pl.ds(d * _LANES, _LANES)
                            acc = None
                            for c in range(_CTX):
                                x = buf[t * _CTX + c, sl]
                                acc = x if acc is None else acc + x
                            urow = ubuf[lrow, sl]
                            term = acc * urow
                            p = term if p is None else p + term
                        pbuf[mg, pl.ds((_CB * b + t) * _LANES, _LANES)] = \
                            p * scale
                pltpu.async_copy(pbuf.at[mg], p_hbm.at[wid, mg], psem)

    @pl.loop(0, _MROWS)
    def _drain(i):
        pltpu.make_async_copy(pbuf.at[0], p_hbm.at[wid, 0], psem).wait()


@functools.cache
def _sc_pdots():
    return pl.kernel(
        _sc_body,
        out_type=jax.ShapeDtypeStruct((_NW, _MROWS, 8 * _LANES),
                                      jnp.float32),
        mesh=plsc.VectorSubcoreMesh(core_axis_name="c", subcore_axis_name="s",
                                    num_cores=2, num_subcores=16),
        compiler_params=pltpu.CompilerParams(use_tc_tiling_on_sc=True),
        scratch_types=[
            pltpu.VMEM((_NCHUNK, _ROWS), jnp.int32),
            pltpu.VMEM((_NUB, _UBLK), jnp.int32),
            pltpu.VMEM((_UBLK, 2 * _DIM), jnp.float32),
            pltpu.VMEM((_UBLK, 2 * _DIM), jnp.float32),
            pltpu.VMEM((_ROWS, 2 * _DIM), jnp.float32),
            pltpu.VMEM((_ROWS, 2 * _DIM), jnp.float32),
            pltpu.VMEM((_MROWS, 8 * _LANES), jnp.float32),
            pltpu.SemaphoreType.DMA,
            pltpu.SemaphoreType.DMA,
            pltpu.SemaphoreType.DMA,
            pltpu.SemaphoreType.DMA,
            pltpu.SemaphoreType.DMA,
        ],
    )


def _loss_body(p_ref, out_ref):
    x = p_ref[...]
    j = lax.broadcasted_iota(jnp.int32, (8 * _LANES, 8), 0)
    t = lax.broadcasted_iota(jnp.int32, (8 * _LANES, 8), 1)
    sel = (j // _LANES == t).astype(jnp.float32)
    z = jnp.dot(x, sel, preferred_element_type=jnp.float32)
    out_ref[0, 0] = -jnp.sum(jax.nn.log_sigmoid(z))


_loss_call = pl.pallas_call(
    _loss_body,
    out_shape=jax.ShapeDtypeStruct((1, 1), jnp.float32),
    out_specs=pl.BlockSpec(memory_space=pltpu.SMEM),
)


def kernel(pos_v, pos_u, neg_v, neg_u, v_table, u_table):
    vidx = jnp.concatenate([pos_v.astype(jnp.int32).reshape(-1),
                            neg_v.astype(jnp.int32).reshape(-1)])
    uidx = jnp.concatenate([pos_u.astype(jnp.int32),
                            neg_u.astype(jnp.int32)])
    vdiv = vidx.reshape(_NW, _NCHUNK, _ROWS)
    udiv = uidx.reshape(_NW, _NUB, _UBLK)
    vpad = jnp.pad(v_table, ((0, 0), (0, _DIM)))
    upad = jnp.pad(u_table, ((0, 0), (0, _DIM)))
    p = _sc_pdots()(vdiv, udiv, vpad, upad)
    loss = _loss_call(p.reshape(_TASKS // 8, 8 * _LANES))
    return loss[0, 0]


# speedup vs baseline: 1.2657x; 1.1139x over previous
import functools

import jax
import jax.numpy as jnp
from jax import lax
from jax.experimental import pallas as pl
from jax.experimental.pallas import tpu as pltpu
from jax.experimental.pallas import tpu_sc as plsc

_VOCAB = 1000000
_DIM = 64
_B = 16384
_CTX = 20
_LANES = 16

_NW = 32
_TASKS = 2 * _B
_TPW = _TASKS // _NW
_CB = 4
_ROWS = _CB * _CTX
_NCHUNK = _TPW // _CB
_UBLK = 32
_NUB = _TPW // _UBLK
_CPU = _UBLK // (2 * _CB)
_MROWS = _TPW // 8


def _sc_body(vdiv_hbm, udiv_hbm, wtab_hbm, p_hbm,
             vdiv_v, udiv_v, ubuf0, ubuf1,
             vbuf0, vbuf1, pbuf, vsem0, vsem1, usem0, usem1, psem):
    cid = lax.axis_index("c")
    sid = lax.axis_index("s")
    wid = sid * 2 + cid

    pltpu.sync_copy(vdiv_hbm.at[wid], vdiv_v)
    pltpu.sync_copy(udiv_hbm.at[wid], udiv_v)

    pltpu.async_copy(wtab_hbm.at[udiv_v.at[0]], ubuf0, usem0)
    pltpu.async_copy(wtab_hbm.at[vdiv_v.at[0]], vbuf0, vsem0)

    sign = jnp.where(wid < _NW // 2, 1.0, -1.0).astype(jnp.float32)
    scale = sign * (1.0 / _CTX)

    vbufs = (vbuf0, vbuf1)
    vsems = (vsem0, vsem1)
    ubufs = (ubuf0, ubuf1)
    usems = (usem0, usem1)

    @pl.loop(0, _NUB // 2)
    def _outer(uu):
        for u2 in range(2):
            ublk = uu * 2 + u2

            pltpu.make_async_copy(wtab_hbm.at[udiv_v.at[ublk]],
                                  ubufs[u2], usems[u2]).wait()

            @pl.when(ublk + 1 < _NUB)
            def _():
                nb = jnp.minimum(ublk + 1, _NUB - 1)
                pltpu.async_copy(wtab_hbm.at[udiv_v.at[nb]],
                                 ubufs[1 - u2], usems[1 - u2])

            ubuf = ubufs[u2]

            @pl.loop(0, _CPU)
            def _inner(cc):
                mg = ublk * _CPU + cc
                for b in range(2):
                    chunk = mg * 2 + b
                    nxt = chunk + 1

                    @pl.when(nxt < _NCHUNK)
                    def _():
                        nrow = jnp.minimum(nxt, _NCHUNK - 1)
                        pltpu.async_copy(wtab_hbm.at[vdiv_v.at[nrow]],
                                         vbufs[1 - b], vsems[1 - b])

                    pltpu.make_async_copy(wtab_hbm.at[vdiv_v.at[chunk]],
                                          vbufs[b], vsems[b]).wait()

                    buf = vbufs[b]
                    for t in range(_CB):
                        lrow = (cc * 2 + b) * _CB + t
                        p = None
                        for d in range(_DIM // _LANES):
                            sl = pl.ds(d * _LANES, _LANES)
                            acc = None
                            for c in range(_CTX):
                                x = buf[t * _CTX + c, sl]
                                acc = x if acc is None else acc + x
                            urow = ubuf[lrow,
                                        pl.ds(_DIM + d * _LANES, _LANES)]
                            term = acc * urow
                            p = term if p is None else p + term
                        pbuf[mg, pl.ds((_CB * b + t) * _LANES, _LANES)] = \
                            p * scale
                pltpu.async_copy(pbuf.at[mg], p_hbm.at[wid, mg], psem)

    @pl.loop(0, _MROWS)
    def _drain(i):
        pltpu.make_async_copy(pbuf.at[0], p_hbm.at[wid, 0], psem).wait()


@functools.cache
def _sc_pdots():
    return pl.kernel(
        _sc_body,
        out_type=jax.ShapeDtypeStruct((_NW, _MROWS, 8 * _LANES),
                                      jnp.float32),
        mesh=plsc.VectorSubcoreMesh(core_axis_name="c", subcore_axis_name="s",
                                    num_cores=2, num_subcores=16),
        compiler_params=pltpu.CompilerParams(use_tc_tiling_on_sc=True),
        scratch_types=[
            pltpu.VMEM((_NCHUNK, _ROWS), jnp.int32),
            pltpu.VMEM((_NUB, _UBLK), jnp.int32),
            pltpu.VMEM((_UBLK, 2 * _DIM), jnp.float32),
            pltpu.VMEM((_UBLK, 2 * _DIM), jnp.float32),
            pltpu.VMEM((_ROWS, 2 * _DIM), jnp.float32),
            pltpu.VMEM((_ROWS, 2 * _DIM), jnp.float32),
            pltpu.VMEM((_MROWS, 8 * _LANES), jnp.float32),
            pltpu.SemaphoreType.DMA,
            pltpu.SemaphoreType.DMA,
            pltpu.SemaphoreType.DMA,
            pltpu.SemaphoreType.DMA,
            pltpu.SemaphoreType.DMA,
        ],
    )


def _loss_body(p_ref, out_ref):
    x = p_ref[...]
    j = lax.broadcasted_iota(jnp.int32, (8 * _LANES, 8), 0)
    t = lax.broadcasted_iota(jnp.int32, (8 * _LANES, 8), 1)
    sel = (j // _LANES == t).astype(jnp.float32)
    z = jnp.dot(x, sel, preferred_element_type=jnp.float32)
    out_ref[0, 0] = -jnp.sum(jax.nn.log_sigmoid(z))


_loss_call = pl.pallas_call(
    _loss_body,
    out_shape=jax.ShapeDtypeStruct((1, 1), jnp.float32),
    out_specs=pl.BlockSpec(memory_space=pltpu.SMEM),
)


def kernel(pos_v, pos_u, neg_v, neg_u, v_table, u_table):
    vidx = jnp.concatenate([pos_v.astype(jnp.int32).reshape(-1),
                            neg_v.astype(jnp.int32).reshape(-1)])
    uidx = jnp.concatenate([pos_u.astype(jnp.int32),
                            neg_u.astype(jnp.int32)])
    vdiv = vidx.reshape(_NW, _NCHUNK, _ROWS)
    udiv = uidx.reshape(_NW, _NUB, _UBLK)
    wtab = jnp.concatenate([v_table, u_table], axis=1)
    p = _sc_pdots()(vdiv, udiv, wtab)
    loss = _loss_call(p.reshape(_TASKS // 8, 8 * _LANES))
    return loss[0, 0]
